# Initial kernel scaffold; baseline (speedup 1.0000x reference)
import jax
import jax.numpy as jnp
from jax.experimental import pallas as pl


def kernel(atom_features, pair_features, membership):
    raise NotImplementedError("write your pallas kernel here")


# speedup vs baseline: 3.3440x; 3.3440x over previous
import functools

import jax
import jax.numpy as jnp
from jax import lax
from jax.experimental import pallas as pl
from jax.experimental.pallas import tpu as pltpu
from jax.experimental.pallas import tpu_sc as plsc

N = 320000
D = 128
B = 1024
NC, NS, L = 2, 16, 16
NW = NC * NS
SPW = B // NW
R = 256
PAD = 16
_NBLK = N // PAD
_BITS = 15


def _lower_bound(mem_hbm, sbuf, sem, target):
    def body(_, c):
        lo, hi = c
        active = lo < hi
        mid = lax.div(lo + hi, 2)
        base = jnp.minimum(mid * PAD, N - PAD)
        pltpu.async_copy(mem_hbm.at[pl.ds(base, PAD)], sbuf, sem).wait()
        big = sbuf[:][0] >= target
        lo2 = jnp.where(big, lo, mid + 1)
        hi2 = jnp.where(big, mid, hi)
        return (jnp.where(active, lo2, lo), jnp.where(active, hi2, hi))

    kb, _ = lax.fori_loop(0, _BITS, body, (jnp.int32(0), jnp.int32(_NBLK)))
    base = jnp.maximum(kb - 1, 0) * PAD
    pltpu.async_copy(mem_hbm.at[pl.ds(base, PAD)], sbuf, sem).wait()
    v = sbuf[:]
    cnt = jnp.sum(jnp.where(v < target, jnp.int32(1), jnp.int32(0)))
    return base + cnt


def _sc_body(atom_hbm, mem_hbm, out_hbm, fbuf, mbuf, sbuf,
             acc, sem_f, sem_m, sem_s):
    c = lax.axis_index("c")
    s = lax.axis_index("s")
    wid = s * NC + c
    seg0 = (wid * SPW).astype(jnp.int32)

    start = _lower_bound(mem_hbm, sbuf, sem_s, seg0)
    end = _lower_bound(mem_hbm, sbuf, sem_s, seg0 + SPW)

    zero = jnp.zeros((L,), jnp.float32)
    ninf = jnp.full((L,), -jnp.inf, jnp.float32)

    def init_seg(i, _):
        for j in range(D // L):
            acc[i, L * j:L * (j + 1)] = zero
            acc[i, D + L * j:D + L * (j + 1)] = ninf
        return 0

    lax.fori_loop(0, SPW, init_seg, 0)

    n = end - start
    nb = lax.div(n + (R - 1), R)

    def block(g, _):
        lo_g = start + g * R
        hi_g = jnp.minimum(lo_g + R, end)
        off = jnp.minimum(lo_g, N - R)
        pltpu.async_copy(atom_hbm.at[pl.ds(off, R), :], fbuf, sem_f).wait()
        moff = lax.div(off, PAD) * PAD
        pltpu.async_copy(mem_hbm.at[pl.ds(moff, R + PAD)], mbuf, sem_m).wait()
        sh = off - moff
        a2 = lo_g - off
        b2 = hi_g - off

        def tile(t, _):
            r0 = t * L
            mv = mbuf[pl.ds(sh + r0, L)]
            for j in range(L):
                r = r0 + j
                idx = mv[j] - seg0

                @pl.when((r >= a2) & (r < b2))
                def _():
                    for q in range(D // L):
                        f = fbuf[r, L * q:L * (q + 1)]
                        acc[idx, L * q:L * (q + 1)] += f
                        cur = acc[idx, D + L * q:D + L * (q + 1)]
                        acc[idx, D + L * q:D + L * (q + 1)] = jnp.maximum(cur, f)

            return 0

        lax.fori_loop(0, R // L, tile, 0)
        return 0

    lax.fori_loop(0, nb, block, 0)

    pltpu.sync_copy(acc, out_hbm.at[pl.ds(seg0, SPW), :])


@jax.jit
def _graph_gather(atom_features, mem_padded):
    mesh = plsc.VectorSubcoreMesh(core_axis_name="c", subcore_axis_name="s")
    return pl.kernel(
        _sc_body,
        out_type=jax.ShapeDtypeStruct((B, 2 * D), jnp.float32),
        mesh=mesh,
        compiler_params=pltpu.CompilerParams(
            use_tc_tiling_on_sc=False, needs_layout_passes=False),
        scratch_types=[
            pltpu.VMEM((R, D), jnp.float32),
            pltpu.VMEM((R + PAD,), jnp.int32),
            pltpu.VMEM((PAD,), jnp.int32),
            pltpu.VMEM((SPW, 2 * D), jnp.float32),
            pltpu.SemaphoreType.DMA,
            pltpu.SemaphoreType.DMA,
            pltpu.SemaphoreType.DMA,
        ],
    )(atom_features, mem_padded)


def kernel(atom_features, pair_features, membership):
    del pair_features
    mem = membership.astype(jnp.int32)
    mem_padded = jnp.concatenate(
        [mem, jnp.full((PAD,), jnp.int32(B), dtype=jnp.int32)])
    return _graph_gather(atom_features, mem_padded)


# speedup vs baseline: 4.0512x; 1.2115x over previous
import functools

import jax
import jax.numpy as jnp
from jax import lax
from jax.experimental import pallas as pl
from jax.experimental.pallas import tpu as pltpu
from jax.experimental.pallas import tpu_sc as plsc

N = 320000
D = 128
B = 1024
NC, NS, L = 2, 16, 16
NW = NC * NS
SPW = B // NW
R = 256
PAD = 16
_NBLK = N // PAD
_BITS = 15


def _lower_bound(mem_hbm, sbuf, sem, target):
    def body(_, c):
        lo, hi = c
        active = lo < hi
        mid = lax.div(lo + hi, 2)
        base = jnp.minimum(mid * PAD, N - PAD)
        pltpu.async_copy(mem_hbm.at[pl.ds(base, PAD)], sbuf, sem).wait()
        big = sbuf[:][0] >= target
        lo2 = jnp.where(big, lo, mid + 1)
        hi2 = jnp.where(big, mid, hi)
        return (jnp.where(active, lo2, lo), jnp.where(active, hi2, hi))

    kb, _ = lax.fori_loop(0, _BITS, body, (jnp.int32(0), jnp.int32(_NBLK)))
    base = jnp.maximum(kb - 1, 0) * PAD
    pltpu.async_copy(mem_hbm.at[pl.ds(base, PAD)], sbuf, sem).wait()
    v = sbuf[:]
    cnt = jnp.sum(jnp.where(v < target, jnp.int32(1), jnp.int32(0)))
    return base + cnt


def _sc_body(atom_hbm, mem_hbm, out_hbm, fbuf, mbuf, sbuf,
             acc, sem_f, sem_m, sem_s):
    c = lax.axis_index("c")
    s = lax.axis_index("s")
    wid = s * NC + c
    seg0 = (wid * SPW).astype(jnp.int32)

    start = _lower_bound(mem_hbm, sbuf, sem_s, seg0)
    end = _lower_bound(mem_hbm, sbuf, sem_s, seg0 + SPW)

    zero = jnp.zeros((L,), jnp.float32)
    ninf = jnp.full((L,), -jnp.inf, jnp.float32)

    def init_seg(i, _):
        for j in range(D // L):
            acc[i, L * j:L * (j + 1)] = zero
            acc[i, D + L * j:D + L * (j + 1)] = ninf
        return 0

    lax.fori_loop(0, SPW, init_seg, 0)

    n = end - start
    nb = lax.div(n + (R - 1), R)

    def offsets(g):
        lo_g = start + g * R
        off = jnp.minimum(lo_g, N - R)
        moff = lax.div(off, PAD) * PAD
        return lo_g, off, moff

    def issue(g):
        p = g & 1
        _, off, moff = offsets(g)
        pltpu.async_copy(atom_hbm.at[pl.ds(off, R), :], fbuf.at[p], sem_f.at[p])
        pltpu.async_copy(mem_hbm.at[pl.ds(moff, R + PAD)], mbuf.at[p], sem_m.at[p])

    @pl.when(nb > 0)
    def _():
        issue(jnp.int32(0))

    def block(g, _):
        p = g & 1
        lo_g, off, moff = offsets(g)
        hi_g = jnp.minimum(lo_g + R, end)
        pltpu.make_async_copy(
            atom_hbm.at[pl.ds(off, R), :], fbuf.at[p], sem_f.at[p]).wait()
        pltpu.make_async_copy(
            mem_hbm.at[pl.ds(moff, R + PAD)], mbuf.at[p], sem_m.at[p]).wait()

        @pl.when(g + 1 < nb)
        def _():
            issue(g + 1)

        sh = off - moff
        a2 = lo_g - off
        b2 = hi_g - off

        def tile(t, _):
            r0 = t * L
            mv = mbuf[p, pl.ds(sh + r0, L)]
            for j in range(L):
                r = r0 + j
                idx = mv[j] - seg0

                @pl.when((r >= a2) & (r < b2))
                def _():
                    for q in range(D // L):
                        f = fbuf[p, r, L * q:L * (q + 1)]
                        acc[idx, L * q:L * (q + 1)] += f
                        cur = acc[idx, D + L * q:D + L * (q + 1)]
                        acc[idx, D + L * q:D + L * (q + 1)] = jnp.maximum(cur, f)

            return 0

        lax.fori_loop(0, R // L, tile, 0)
        return 0

    lax.fori_loop(0, nb, block, 0)

    pltpu.sync_copy(acc, out_hbm.at[pl.ds(seg0, SPW), :])


@jax.jit
def _graph_gather(atom_features, mem_padded):
    mesh = plsc.VectorSubcoreMesh(core_axis_name="c", subcore_axis_name="s")
    return pl.kernel(
        _sc_body,
        out_type=jax.ShapeDtypeStruct((B, 2 * D), jnp.float32),
        mesh=mesh,
        compiler_params=pltpu.CompilerParams(
            use_tc_tiling_on_sc=False, needs_layout_passes=False),
        scratch_types=[
            pltpu.VMEM((2, R, D), jnp.float32),
            pltpu.VMEM((2, R + PAD), jnp.int32),
            pltpu.VMEM((PAD,), jnp.int32),
            pltpu.VMEM((SPW, 2 * D), jnp.float32),
            pltpu.SemaphoreType.DMA((2,)),
            pltpu.SemaphoreType.DMA((2,)),
            pltpu.SemaphoreType.DMA,
        ],
    )(atom_features, mem_padded)


def kernel(atom_features, pair_features, membership):
    del pair_features
    mem = membership.astype(jnp.int32)
    mem_padded = jnp.concatenate(
        [mem, jnp.full((PAD,), jnp.int32(B), dtype=jnp.int32)])
    return _graph_gather(atom_features, mem_padded)


# speedup vs baseline: 15.2442x; 3.7629x over previous
import functools

import jax
import jax.numpy as jnp
from jax import lax
from jax.experimental import pallas as pl
from jax.experimental.pallas import tpu as pltpu
from jax.experimental.pallas import tpu_sc as plsc

N = 320000
D = 128
B = 1024
NC, NS, L = 2, 16, 16
NW = NC * NS
SPW = B // NW
R = 256
PAD = 16
NQ = D // L
_NBLK = N // PAD
_BITS = 15
BIG = N + 1


def _cummin(v):
    return -plsc.cummax(-v)


def _sufmin(v):
    return lax.rev(_cummin(lax.rev(v, (0,))), (0,))


def _lower_bound(mem_hbm, sbuf, sem, target):
    def body(_, c):
        lo, hi = c
        active = lo < hi
        mid = lax.div(lo + hi, 2)
        base = jnp.minimum(mid * PAD, N - PAD)
        pltpu.async_copy(mem_hbm.at[pl.ds(base, PAD)], sbuf, sem).wait()
        big = sbuf[:][0] >= target
        lo2 = jnp.where(big, lo, mid + 1)
        hi2 = jnp.where(big, mid, hi)
        return (jnp.where(active, lo2, lo), jnp.where(active, hi2, hi))

    kb, _ = lax.fori_loop(0, _BITS, body, (jnp.int32(0), jnp.int32(_NBLK)))
    base = jnp.maximum(kb - 1, 0) * PAD
    pltpu.async_copy(mem_hbm.at[pl.ds(base, PAD)], sbuf, sem).wait()
    v = sbuf[:]
    cnt = jnp.sum(jnp.where(v < target, jnp.int32(1), jnp.int32(0)))
    return base + cnt


def _sc_body(atom_hbm, mem_hbm, out_hbm, fbuf, mbuf, sbuf, bnd, acc,
             sem_f, sem_m, sem_s):
    c = lax.axis_index("c")
    s = lax.axis_index("s")
    wid = s * NC + c
    seg0 = (wid * SPW).astype(jnp.int32)

    start = _lower_bound(mem_hbm, sbuf, sem_s, seg0)
    end = _lower_bound(mem_hbm, sbuf, sem_s, seg0 + SPW)

    zero = jnp.zeros((L,), jnp.float32)
    ninf = jnp.full((L,), -jnp.inf, jnp.float32)
    iota = lax.iota(jnp.int32, L)

    def init_seg(i, _):
        for j in range(NQ):
            acc[i, L * j:L * (j + 1)] = zero
            acc[i, D + L * j:D + L * (j + 1)] = ninf
        return 0

    lax.fori_loop(0, SPW, init_seg, 0)

    bnd[0:L] = jnp.full((L,), BIG, jnp.int32)
    bnd[L:2 * L] = jnp.full((L,), BIG, jnp.int32)
    bnd[2 * L:3 * L] = jnp.where(iota == 0, end, BIG)

    n = end - start
    nb = lax.div(n + (R - 1), R)

    def offsets(g):
        lo_g = start + g * R
        off = jnp.minimum(lo_g, N - R)
        moff = lax.div(off, PAD) * PAD - PAD
        moff = pl.multiple_of(jnp.maximum(moff, 0), PAD)
        return lo_g, off, moff

    def issue(g):
        p = g & 1
        _, off, moff = offsets(g)
        pltpu.async_copy(atom_hbm.at[pl.ds(off, R), :], fbuf.at[p], sem_f.at[p])
        pltpu.async_copy(mem_hbm.at[pl.ds(moff, R + 2 * PAD)], mbuf.at[p],
                         sem_m.at[p])

    @pl.when(nb > 0)
    def _():
        issue(jnp.int32(0))

    def block(g, carry):
        si, *vregs = carry
        p = g & 1
        lo_g, off, moff = offsets(g)
        hi_g = jnp.minimum(lo_g + R, end)
        pltpu.make_async_copy(
            atom_hbm.at[pl.ds(off, R), :], fbuf.at[p], sem_f.at[p]).wait()
        pltpu.make_async_copy(
            mem_hbm.at[pl.ds(moff, R + 2 * PAD)], mbuf.at[p], sem_m.at[p]).wait()

        @pl.when(g + 1 < nb)
        def _():
            issue(g + 1)

        sh = off - moff
        a2 = lo_g - off
        b2 = hi_g - off

        def scan_tile(t, _):
            r0 = t * L
            mv = mbuf[p, pl.ds(sh + r0, L)]
            prev = mbuf[p, pl.ds(jnp.maximum(sh + r0 - 1, 0), L)]
            r_loc = r0 + iota
            mask = ((mv != prev) | (off + r_loc == start)) \
                & (r_loc >= a2) & (r_loc < b2)
            plsc.store_scatter(bnd, [mv - seg0], off + r_loc, mask=mask)
            return 0

        lax.fori_loop(0, R // L, scan_tile, 0)

        t2 = _sufmin(bnd[2 * L:3 * L])
        t1 = jnp.minimum(_sufmin(bnd[L:2 * L]), jnp.broadcast_to(t2[0], (L,)))
        t0 = jnp.minimum(_sufmin(bnd[0:L]), jnp.broadcast_to(t1[0], (L,)))
        bnd[0:L] = t0
        bnd[L:2 * L] = t1
        bnd[2 * L:3 * L] = t2

        w0 = jnp.where((t0 <= hi_g) & (iota >= 1), jnp.int32(1), jnp.int32(0))
        w1 = jnp.where(t1 <= hi_g, jnp.int32(1), jnp.int32(0))
        w2 = jnp.where((t2 <= hi_g) & (iota == 0), jnp.int32(1), jnp.int32(0))
        cnt = jnp.sum(w0 + w1 + w2)
        trips = cnt - si + jnp.where(hi_g < end, jnp.int32(1), jnp.int32(0))

        def run(_, rc):
            si_r, pos_l, *vr = rc
            sv = vr[:NQ]
            mv_ = vr[NQ:]
            nxt = plsc.load_gather(
                bnd, [jnp.full((L,), si_r + 1, jnp.int32)])[0]
            re_l = jnp.minimum(nxt, hi_g) - off

            def row(r, rcv):
                svi = rcv[:NQ]
                mvi = rcv[NQ:]
                out = []
                for q in range(NQ):
                    f = fbuf[p, r, L * q:L * (q + 1)]
                    out.append(svi[q] + f)
                for q in range(NQ):
                    f = fbuf[p, r, L * q:L * (q + 1)]
                    out.append(jnp.maximum(mvi[q], f))
                return out

            vr2 = lax.fori_loop(pos_l, re_l, row, list(sv) + list(mv_))
            sv2 = vr2[:NQ]
            mv2 = vr2[NQ:]
            ended = nxt <= hi_g

            @pl.when(ended)
            def _():
                for q in range(NQ):
                    acc[si_r, L * q:L * (q + 1)] = sv2[q]
                    acc[si_r, D + L * q:D + L * (q + 1)] = mv2[q]

            si2 = jnp.where(ended, si_r + 1, si_r)
            sv3 = [jnp.where(ended, zero, x) for x in sv2]
            mv3 = [jnp.where(ended, ninf, x) for x in mv2]
            return [si2, re_l] + sv3 + mv3

        rc = lax.fori_loop(0, trips, run,
                           [si, a2] + list(vregs))
        return [rc[0]] + rc[2:]

    init_carry = [jnp.int32(0)] + [zero] * NQ + [ninf] * NQ
    lax.fori_loop(0, nb, block, init_carry)

    pltpu.sync_copy(acc, out_hbm.at[pl.ds(seg0, SPW), :])


@jax.jit
def _graph_gather(atom_features, mem_padded):
    mesh = plsc.VectorSubcoreMesh(core_axis_name="c", subcore_axis_name="s")
    return pl.kernel(
        _sc_body,
        out_type=jax.ShapeDtypeStruct((B, 2 * D), jnp.float32),
        mesh=mesh,
        compiler_params=pltpu.CompilerParams(
            use_tc_tiling_on_sc=False, needs_layout_passes=False),
        scratch_types=[
            pltpu.VMEM((2, R, D), jnp.float32),
            pltpu.VMEM((2, R + 2 * PAD), jnp.int32),
            pltpu.VMEM((PAD,), jnp.int32),
            pltpu.VMEM((3 * L,), jnp.int32),
            pltpu.VMEM((SPW, 2 * D), jnp.float32),
            pltpu.SemaphoreType.DMA((2,)),
            pltpu.SemaphoreType.DMA((2,)),
            pltpu.SemaphoreType.DMA,
        ],
    )(atom_features, mem_padded)


def kernel(atom_features, pair_features, membership):
    del pair_features
    mem = membership.astype(jnp.int32)
    mem_padded = jnp.concatenate(
        [mem, jnp.full((PAD,), jnp.int32(B), dtype=jnp.int32)])
    return _graph_gather(atom_features, mem_padded)


# speedup vs baseline: 19.4356x; 1.2749x over previous
import functools

import jax
import jax.numpy as jnp
from jax import lax
from jax.experimental import pallas as pl
from jax.experimental.pallas import tpu as pltpu
from jax.experimental.pallas import tpu_sc as plsc

N = 320000
D = 128
B = 1024
NC, NS, L = 2, 16, 16
NW = NC * NS
SPW = B // NW
R = 256
PAD = 16
NQ = D // L
_NBLK = N // PAD
_BITS = 15
BIG = N + 1


def _cummin(v):
    return -plsc.cummax(-v)


def _sufmin(v):
    return lax.rev(_cummin(lax.rev(v, (0,))), (0,))


def _lower_bound2(mem_hbm, sb1, sb2, sem1, sem2, t1, t2):
    def probe(c, sbuf, sem):
        lo, hi = c
        mid = lax.div(lo + hi, 2)
        base = jnp.minimum(mid * PAD, N - PAD)
        return pltpu.async_copy(mem_hbm.at[pl.ds(base, PAD)], sbuf, sem)

    def upd(c, target, sbuf):
        lo, hi = c
        active = lo < hi
        mid = lax.div(lo + hi, 2)
        big = sbuf[:][0] >= target
        lo2 = jnp.where(big, lo, mid + 1)
        hi2 = jnp.where(big, mid, hi)
        return (jnp.where(active, lo2, lo), jnp.where(active, hi2, hi))

    def body(_, c):
        c1, c2 = c
        d1 = probe(c1, sb1, sem1)
        d2 = probe(c2, sb2, sem2)
        d1.wait()
        d2.wait()
        return (upd(c1, t1, sb1), upd(c2, t2, sb2))

    init = (jnp.int32(0), jnp.int32(_NBLK))
    (kb1, _), (kb2, _) = lax.fori_loop(0, _BITS, body, (init, init))
    base1 = jnp.maximum(kb1 - 1, 0) * PAD
    base2 = jnp.maximum(kb2 - 1, 0) * PAD
    d1 = pltpu.async_copy(mem_hbm.at[pl.ds(base1, PAD)], sb1, sem1)
    d2 = pltpu.async_copy(mem_hbm.at[pl.ds(base2, PAD)], sb2, sem2)
    d1.wait()
    d2.wait()
    cnt1 = jnp.sum(jnp.where(sb1[:] < t1, jnp.int32(1), jnp.int32(0)))
    cnt2 = jnp.sum(jnp.where(sb2[:] < t2, jnp.int32(1), jnp.int32(0)))
    return base1 + cnt1, base2 + cnt2


def _sc_body(atom_hbm, mem_hbm, out_hbm, fbuf, mbuf, sbuf, sbuf2, bnd, acc,
             sem_f, sem_m, sem_s, sem_s2):
    c = lax.axis_index("c")
    s = lax.axis_index("s")
    wid = s * NC + c
    seg0 = (wid * SPW).astype(jnp.int32)

    start, end = _lower_bound2(mem_hbm, sbuf, sbuf2, sem_s, sem_s2,
                               seg0, seg0 + SPW)

    zero = jnp.zeros((L,), jnp.float32)
    ninf = jnp.full((L,), -jnp.inf, jnp.float32)
    iota = lax.iota(jnp.int32, L)

    def init_seg(i, _):
        for j in range(NQ):
            acc[i, L * j:L * (j + 1)] = zero
            acc[i, D + L * j:D + L * (j + 1)] = ninf
        return 0

    lax.fori_loop(0, SPW, init_seg, 0)

    bnd[0:L] = jnp.full((L,), BIG, jnp.int32)
    bnd[L:2 * L] = jnp.full((L,), BIG, jnp.int32)
    bnd[2 * L:3 * L] = jnp.where(iota == 0, end, BIG)

    n = end - start
    nb = lax.div(n + (R - 1), R)

    def offsets(g):
        lo_g = start + g * R
        off = jnp.minimum(lo_g, N - R)
        moff = lax.div(off, PAD) * PAD - PAD
        moff = pl.multiple_of(jnp.maximum(moff, 0), PAD)
        return lo_g, off, moff

    def issue(g, p):
        _, off, moff = offsets(g)
        pltpu.async_copy(atom_hbm.at[pl.ds(off, R), :], fbuf.at[p], sem_f.at[p])
        pltpu.async_copy(mem_hbm.at[pl.ds(moff, R + 2 * PAD)], mbuf.at[p],
                         sem_m.at[p])

    @pl.when(nb > 0)
    def _():
        issue(jnp.int32(0), jnp.int32(0))

    @pl.when(nb > 1)
    def _():
        issue(jnp.int32(1), jnp.int32(1))

    def block(g, carry):
        si, p, *vregs = carry
        lo_g, off, moff = offsets(g)
        hi_g = jnp.minimum(lo_g + R, end)
        pltpu.make_async_copy(
            atom_hbm.at[pl.ds(off, R), :], fbuf.at[p], sem_f.at[p]).wait()
        pltpu.make_async_copy(
            mem_hbm.at[pl.ds(moff, R + 2 * PAD)], mbuf.at[p], sem_m.at[p]).wait()

        @pl.when(g + 2 < nb)
        def _():
            p2 = jnp.where(p == 0, jnp.int32(2), p - 1)
            issue(g + 2, p2)

        sh = off - moff
        a2 = lo_g - off
        b2 = hi_g - off

        def scan_tile(t, _):
            r0 = t * L
            mv = mbuf[p, pl.ds(sh + r0, L)]
            prev = mbuf[p, pl.ds(jnp.maximum(sh + r0 - 1, 0), L)]
            r_loc = r0 + iota
            mask = ((mv != prev) | (off + r_loc == start)) \
                & (r_loc >= a2) & (r_loc < b2)
            plsc.store_scatter(bnd, [mv - seg0], off + r_loc, mask=mask)
            return 0

        lax.fori_loop(0, R // L, scan_tile, 0)

        t2 = _sufmin(bnd[2 * L:3 * L])
        t1 = jnp.minimum(_sufmin(bnd[L:2 * L]), jnp.broadcast_to(t2[0], (L,)))
        t0 = jnp.minimum(_sufmin(bnd[0:L]), jnp.broadcast_to(t1[0], (L,)))
        bnd[0:L] = t0
        bnd[L:2 * L] = t1
        bnd[2 * L:3 * L] = t2

        w0 = jnp.where((t0 <= hi_g) & (iota >= 1), jnp.int32(1), jnp.int32(0))
        w1 = jnp.where(t1 <= hi_g, jnp.int32(1), jnp.int32(0))
        w2 = jnp.where((t2 <= hi_g) & (iota == 0), jnp.int32(1), jnp.int32(0))
        cnt = jnp.sum(w0 + w1 + w2)
        trips = cnt - si + jnp.where(hi_g < end, jnp.int32(1), jnp.int32(0))

        def run(_, rc):
            si_r, pos_l, *vr = rc
            sv = vr[:NQ]
            mv_ = vr[NQ:]
            nxt = plsc.load_gather(
                bnd, [jnp.full((L,), si_r + 1, jnp.int32)])[0]
            re_l = jnp.minimum(nxt, hi_g) - off

            def row(r, rcv):
                svi = rcv[:NQ]
                mvi = rcv[NQ:]
                out = []
                for q in range(NQ):
                    f = fbuf[p, r, L * q:L * (q + 1)]
                    out.append(svi[q] + f)
                for q in range(NQ):
                    f = fbuf[p, r, L * q:L * (q + 1)]
                    out.append(jnp.maximum(mvi[q], f))
                return out

            vr2 = lax.fori_loop(pos_l, re_l, row, list(sv) + list(mv_))
            sv2 = vr2[:NQ]
            mv2 = vr2[NQ:]
            ended = nxt <= hi_g

            @pl.when(ended)
            def _():
                for q in range(NQ):
                    acc[si_r, L * q:L * (q + 1)] = sv2[q]
                    acc[si_r, D + L * q:D + L * (q + 1)] = mv2[q]

            si2 = jnp.where(ended, si_r + 1, si_r)
            sv3 = [jnp.where(ended, zero, x) for x in sv2]
            mv3 = [jnp.where(ended, ninf, x) for x in mv2]
            return [si2, re_l] + sv3 + mv3

        rc = lax.fori_loop(0, trips, run,
                           [si, a2] + list(vregs))
        p_next = jnp.where(p == 2, jnp.int32(0), p + 1)
        return [rc[0], p_next] + rc[2:]

    init_carry = [jnp.int32(0), jnp.int32(0)] + [zero] * NQ + [ninf] * NQ
    lax.fori_loop(0, nb, block, init_carry)

    pltpu.sync_copy(acc, out_hbm.at[pl.ds(seg0, SPW), :])


@jax.jit
def _graph_gather(atom_features, mem_padded):
    mesh = plsc.VectorSubcoreMesh(core_axis_name="c", subcore_axis_name="s")
    return pl.kernel(
        _sc_body,
        out_type=jax.ShapeDtypeStruct((B, 2 * D), jnp.float32),
        mesh=mesh,
        compiler_params=pltpu.CompilerParams(
            use_tc_tiling_on_sc=False, needs_layout_passes=False),
        scratch_types=[
            pltpu.VMEM((3, R, D), jnp.float32),
            pltpu.VMEM((3, R + 2 * PAD), jnp.int32),
            pltpu.VMEM((PAD,), jnp.int32),
            pltpu.VMEM((PAD,), jnp.int32),
            pltpu.VMEM((3 * L,), jnp.int32),
            pltpu.VMEM((SPW, 2 * D), jnp.float32),
            pltpu.SemaphoreType.DMA((3,)),
            pltpu.SemaphoreType.DMA((3,)),
            pltpu.SemaphoreType.DMA,
            pltpu.SemaphoreType.DMA,
        ],
    )(atom_features, mem_padded)


def kernel(atom_features, pair_features, membership):
    del pair_features
    mem = membership.astype(jnp.int32)
    mem_padded = jnp.concatenate(
        [mem, jnp.full((PAD,), jnp.int32(B), dtype=jnp.int32)])
    return _graph_gather(atom_features, mem_padded)


# speedup vs baseline: 19.5001x; 1.0033x over previous
import functools

import jax
import jax.numpy as jnp
from jax import lax
from jax.experimental import pallas as pl
from jax.experimental.pallas import tpu as pltpu
from jax.experimental.pallas import tpu_sc as plsc

N = 320000
D = 128
B = 1024
NC, NS, L = 2, 16, 16
NW = NC * NS
SPW = B // NW
R = 256
PAD = 16
NQ = D // L
_NBLK = N // PAD
_BITS = 15
BIG = N + 1


def _cummin(v):
    return -plsc.cummax(-v)


def _sufmin(v):
    return lax.rev(_cummin(lax.rev(v, (0,))), (0,))


def _lower_bound2(mem_hbm, sb1, sb2, sem1, sem2, t1, t2):
    def probe(c, sbuf, sem):
        lo, hi = c
        mid = lax.div(lo + hi, 2)
        base = jnp.minimum(mid * PAD, N - PAD)
        return pltpu.async_copy(mem_hbm.at[pl.ds(base, PAD)], sbuf, sem)

    def upd(c, target, sbuf):
        lo, hi = c
        active = lo < hi
        mid = lax.div(lo + hi, 2)
        big = sbuf[:][0] >= target
        lo2 = jnp.where(big, lo, mid + 1)
        hi2 = jnp.where(big, mid, hi)
        return (jnp.where(active, lo2, lo), jnp.where(active, hi2, hi))

    def body(_, c):
        c1, c2 = c
        d1 = probe(c1, sb1, sem1)
        d2 = probe(c2, sb2, sem2)
        d1.wait()
        d2.wait()
        return (upd(c1, t1, sb1), upd(c2, t2, sb2))

    init = (jnp.int32(0), jnp.int32(_NBLK))
    (kb1, _), (kb2, _) = lax.fori_loop(0, _BITS, body, (init, init))
    base1 = jnp.maximum(kb1 - 1, 0) * PAD
    base2 = jnp.maximum(kb2 - 1, 0) * PAD
    d1 = pltpu.async_copy(mem_hbm.at[pl.ds(base1, PAD)], sb1, sem1)
    d2 = pltpu.async_copy(mem_hbm.at[pl.ds(base2, PAD)], sb2, sem2)
    d1.wait()
    d2.wait()
    cnt1 = jnp.sum(jnp.where(sb1[:] < t1, jnp.int32(1), jnp.int32(0)))
    cnt2 = jnp.sum(jnp.where(sb2[:] < t2, jnp.int32(1), jnp.int32(0)))
    return base1 + cnt1, base2 + cnt2


def _sc_body(atom_hbm, mem_hbm, out_hbm, fbuf, mbuf, sbuf, sbuf2, bnd, acc,
             sem_f, sem_m, sem_s, sem_s2):
    c = lax.axis_index("c")
    s = lax.axis_index("s")
    wid = s * NC + c
    seg0 = (wid * SPW).astype(jnp.int32)

    start, end = _lower_bound2(mem_hbm, sbuf, sbuf2, sem_s, sem_s2,
                               seg0, seg0 + SPW)

    zero = jnp.zeros((L,), jnp.float32)
    ninf = jnp.full((L,), -jnp.inf, jnp.float32)
    iota = lax.iota(jnp.int32, L)

    def init_seg(i, _):
        for j in range(NQ):
            acc[i, L * j:L * (j + 1)] = zero
            acc[i, D + L * j:D + L * (j + 1)] = ninf
        return 0

    lax.fori_loop(0, SPW, init_seg, 0)

    bnd[0:L] = jnp.full((L,), BIG, jnp.int32)
    bnd[L:2 * L] = jnp.full((L,), BIG, jnp.int32)
    bnd[2 * L:3 * L] = jnp.where(iota == 0, end, BIG)

    n = end - start
    nb = lax.div(n + (R - 1), R)

    def offsets(g):
        lo_g = start + g * R
        off = jnp.minimum(lo_g, N - R)
        moff = lax.div(off, PAD) * PAD - PAD
        moff = pl.multiple_of(jnp.maximum(moff, 0), PAD)
        return lo_g, off, moff

    def issue(g, p):
        _, off, moff = offsets(g)
        pltpu.async_copy(atom_hbm.at[pl.ds(off, R), :], fbuf.at[p], sem_f.at[p])
        pltpu.async_copy(mem_hbm.at[pl.ds(moff, R + 2 * PAD)], mbuf.at[p],
                         sem_m.at[p])

    @pl.when(nb > 0)
    def _():
        issue(jnp.int32(0), jnp.int32(0))

    @pl.when(nb > 1)
    def _():
        issue(jnp.int32(1), jnp.int32(1))

    def block(g, carry):
        si, p, *vregs = carry
        lo_g, off, moff = offsets(g)
        hi_g = jnp.minimum(lo_g + R, end)
        pltpu.make_async_copy(
            mem_hbm.at[pl.ds(moff, R + 2 * PAD)], mbuf.at[p], sem_m.at[p]).wait()

        sh = off - moff
        a2 = lo_g - off
        b2 = hi_g - off

        def scan_tile(t, _):
            r0 = t * L
            mv = mbuf[p, pl.ds(sh + r0, L)]
            prev = mbuf[p, pl.ds(jnp.maximum(sh + r0 - 1, 0), L)]
            r_loc = r0 + iota
            mask = ((mv != prev) | (off + r_loc == start)) \
                & (r_loc >= a2) & (r_loc < b2)
            plsc.store_scatter(bnd, [mv - seg0], off + r_loc, mask=mask)
            return 0

        lax.fori_loop(0, R // L, scan_tile, 0)

        t2 = _sufmin(bnd[2 * L:3 * L])
        t1 = jnp.minimum(_sufmin(bnd[L:2 * L]), jnp.broadcast_to(t2[0], (L,)))
        t0 = jnp.minimum(_sufmin(bnd[0:L]), jnp.broadcast_to(t1[0], (L,)))
        bnd[0:L] = t0
        bnd[L:2 * L] = t1
        bnd[2 * L:3 * L] = t2

        w0 = jnp.where((t0 <= hi_g) & (iota >= 1), jnp.int32(1), jnp.int32(0))
        w1 = jnp.where(t1 <= hi_g, jnp.int32(1), jnp.int32(0))
        w2 = jnp.where((t2 <= hi_g) & (iota == 0), jnp.int32(1), jnp.int32(0))
        cnt = jnp.sum(w0 + w1 + w2)
        trips = cnt - si + jnp.where(hi_g < end, jnp.int32(1), jnp.int32(0))

        pltpu.make_async_copy(
            atom_hbm.at[pl.ds(off, R), :], fbuf.at[p], sem_f.at[p]).wait()

        @pl.when(g + 2 < nb)
        def _():
            p2 = jnp.where(p == 0, jnp.int32(2), p - 1)
            issue(g + 2, p2)

        def run(_, rc):
            si_r, pos_l, *vr = rc
            sv = vr[:NQ]
            mv_ = vr[NQ:]
            nxt = plsc.load_gather(
                bnd, [jnp.full((L,), si_r + 1, jnp.int32)])[0]
            re_l = jnp.minimum(nxt, hi_g) - off

            def row(r, rcv):
                svi = rcv[:NQ]
                mvi = rcv[NQ:]
                out = []
                for q in range(NQ):
                    f = fbuf[p, r, L * q:L * (q + 1)]
                    out.append(svi[q] + f)
                for q in range(NQ):
                    f = fbuf[p, r, L * q:L * (q + 1)]
                    out.append(jnp.maximum(mvi[q], f))
                return out

            vr2 = lax.fori_loop(pos_l, re_l, row, list(sv) + list(mv_))
            sv2 = vr2[:NQ]
            mv2 = vr2[NQ:]
            ended = nxt <= hi_g

            @pl.when(ended)
            def _():
                for q in range(NQ):
                    acc[si_r, L * q:L * (q + 1)] = sv2[q]
                    acc[si_r, D + L * q:D + L * (q + 1)] = mv2[q]

            si2 = jnp.where(ended, si_r + 1, si_r)
            sv3 = [jnp.where(ended, zero, x) for x in sv2]
            mv3 = [jnp.where(ended, ninf, x) for x in mv2]
            return [si2, re_l] + sv3 + mv3

        rc = lax.fori_loop(0, trips, run,
                           [si, a2] + list(vregs))
        p_next = jnp.where(p == 2, jnp.int32(0), p + 1)
        return [rc[0], p_next] + rc[2:]

    init_carry = [jnp.int32(0), jnp.int32(0)] + [zero] * NQ + [ninf] * NQ
    lax.fori_loop(0, nb, block, init_carry)

    pltpu.sync_copy(acc, out_hbm.at[pl.ds(seg0, SPW), :])


@jax.jit
def _graph_gather(atom_features, mem_padded):
    mesh = plsc.VectorSubcoreMesh(core_axis_name="c", subcore_axis_name="s")
    return pl.kernel(
        _sc_body,
        out_type=jax.ShapeDtypeStruct((B, 2 * D), jnp.float32),
        mesh=mesh,
        compiler_params=pltpu.CompilerParams(
            use_tc_tiling_on_sc=False, needs_layout_passes=False),
        scratch_types=[
            pltpu.VMEM((3, R, D), jnp.float32),
            pltpu.VMEM((3, R + 2 * PAD), jnp.int32),
            pltpu.VMEM((PAD,), jnp.int32),
            pltpu.VMEM((PAD,), jnp.int32),
            pltpu.VMEM((3 * L,), jnp.int32),
            pltpu.VMEM((SPW, 2 * D), jnp.float32),
            pltpu.SemaphoreType.DMA((3,)),
            pltpu.SemaphoreType.DMA((3,)),
            pltpu.SemaphoreType.DMA,
            pltpu.SemaphoreType.DMA,
        ],
    )(atom_features, mem_padded)


def kernel(atom_features, pair_features, membership):
    del pair_features
    mem = membership.astype(jnp.int32)
    mem_padded = jnp.concatenate(
        [mem, jnp.full((PAD,), jnp.int32(B), dtype=jnp.int32)])
    return _graph_gather(atom_features, mem_padded)


# speedup vs baseline: 20.3873x; 1.0455x over previous
import functools

import jax
import jax.numpy as jnp
from jax import lax
from jax.experimental import pallas as pl
from jax.experimental.pallas import tpu as pltpu
from jax.experimental.pallas import tpu_sc as plsc

N = 320000
D = 128
B = 1024
NC, NS, L = 2, 16, 16
NW = NC * NS
SPW = B // NW
R = 256
PAD = 16
NQ = D // L
_NBLK = N // PAD
_BITS = 15
BIG = N + 1


def _cummin(v):
    return -plsc.cummax(-v)


def _sufmin(v):
    return lax.rev(_cummin(lax.rev(v, (0,))), (0,))


_ROUNDS = 5


def _lower_bound2(mem_hbm, sb1, sb2, sem1, sem2, t1, t2):
    iota = lax.iota(jnp.int32, L)

    def probe(c, sbuf, sem):
        lo, sz = c
        pos = lo + lax.div(sz * iota, L)
        rows = jnp.minimum(pos, _NBLK - 1) * PAD
        return pltpu.async_copy(mem_hbm.at[rows], sbuf, sem), pos

    def upd(c, target, sbuf, pos):
        lo, sz = c
        heads = jnp.where(pos >= _NBLK, jnp.int32(B), sbuf[:])
        cnt = jnp.sum(jnp.where(heads < target, jnp.int32(1), jnp.int32(0)))
        lo2 = jnp.where(cnt == 0, lo, lo + lax.div(sz * (cnt - 1), L) + 1)
        hi2 = jnp.where(cnt >= L, lo + sz,
                        jnp.where(cnt == 0, lo, lo + lax.div(sz * cnt, L)))
        return (lo2, hi2 - lo2)

    def body(_, c):
        c1, c2 = c
        d1, pos1 = probe(c1, sb1, sem1)
        d2, pos2 = probe(c2, sb2, sem2)
        d1.wait()
        d2.wait()
        return (upd(c1, t1, sb1, pos1), upd(c2, t2, sb2, pos2))

    init = (jnp.int32(0), jnp.int32(_NBLK))
    (kb1, _), (kb2, _) = lax.fori_loop(0, _ROUNDS, body, (init, init))
    base1 = jnp.maximum(kb1 - 1, 0) * PAD
    base2 = jnp.maximum(kb2 - 1, 0) * PAD
    d1 = pltpu.async_copy(mem_hbm.at[pl.ds(base1, PAD)], sb1, sem1)
    d2 = pltpu.async_copy(mem_hbm.at[pl.ds(base2, PAD)], sb2, sem2)
    d1.wait()
    d2.wait()
    cnt1 = jnp.sum(jnp.where(sb1[:] < t1, jnp.int32(1), jnp.int32(0)))
    cnt2 = jnp.sum(jnp.where(sb2[:] < t2, jnp.int32(1), jnp.int32(0)))
    return base1 + cnt1, base2 + cnt2


def _sc_body(atom_hbm, mem_hbm, out_hbm, fbuf, mbuf, sbuf, sbuf2, bnd, acc,
             sem_f, sem_m, sem_s, sem_s2):
    c = lax.axis_index("c")
    s = lax.axis_index("s")
    wid = s * NC + c
    seg0 = (wid * SPW).astype(jnp.int32)

    start, end = _lower_bound2(mem_hbm, sbuf, sbuf2, sem_s, sem_s2,
                               seg0, seg0 + SPW)

    zero = jnp.zeros((L,), jnp.float32)
    ninf = jnp.full((L,), -jnp.inf, jnp.float32)
    iota = lax.iota(jnp.int32, L)

    def init_seg(i, _):
        for j in range(NQ):
            acc[i, L * j:L * (j + 1)] = zero
            acc[i, D + L * j:D + L * (j + 1)] = ninf
        return 0

    lax.fori_loop(0, SPW, init_seg, 0)

    bnd[0:L] = jnp.full((L,), BIG, jnp.int32)
    bnd[L:2 * L] = jnp.full((L,), BIG, jnp.int32)
    bnd[2 * L:3 * L] = jnp.where(iota == 0, end, BIG)

    n = end - start
    nb = lax.div(n + (R - 1), R)

    def offsets(g):
        lo_g = start + g * R
        off = jnp.minimum(lo_g, N - R)
        moff = lax.div(off, PAD) * PAD - PAD
        moff = jnp.minimum(moff, N - (R + 2 * PAD))
        moff = pl.multiple_of(jnp.maximum(moff, 0), PAD)
        return lo_g, off, moff

    def issue(g, p):
        _, off, moff = offsets(g)
        pltpu.async_copy(atom_hbm.at[pl.ds(off, R), :], fbuf.at[p], sem_f.at[p])
        pltpu.async_copy(mem_hbm.at[pl.ds(moff, R + 2 * PAD)], mbuf.at[p],
                         sem_m.at[p])

    @pl.when(nb > 0)
    def _():
        issue(jnp.int32(0), jnp.int32(0))

    @pl.when(nb > 1)
    def _():
        issue(jnp.int32(1), jnp.int32(1))

    def block(g, carry):
        si, p, *vregs = carry
        lo_g, off, moff = offsets(g)
        hi_g = jnp.minimum(lo_g + R, end)
        pltpu.make_async_copy(
            mem_hbm.at[pl.ds(moff, R + 2 * PAD)], mbuf.at[p], sem_m.at[p]).wait()

        sh = off - moff
        a2 = lo_g - off
        b2 = hi_g - off

        def scan_tile(t, _):
            r0 = t * L
            mv = mbuf[p, pl.ds(sh + r0, L)]
            prev = mbuf[p, pl.ds(jnp.maximum(sh + r0 - 1, 0), L)]
            r_loc = r0 + iota
            mask = ((mv != prev) | (off + r_loc == start)) \
                & (r_loc >= a2) & (r_loc < b2)
            plsc.store_scatter(bnd, [mv - seg0], off + r_loc, mask=mask)
            return 0

        lax.fori_loop(0, R // L, scan_tile, 0)

        t2 = _sufmin(bnd[2 * L:3 * L])
        t1 = jnp.minimum(_sufmin(bnd[L:2 * L]), jnp.broadcast_to(t2[0], (L,)))
        t0 = jnp.minimum(_sufmin(bnd[0:L]), jnp.broadcast_to(t1[0], (L,)))
        bnd[0:L] = t0
        bnd[L:2 * L] = t1
        bnd[2 * L:3 * L] = t2

        w0 = jnp.where((t0 <= hi_g) & (iota >= 1), jnp.int32(1), jnp.int32(0))
        w1 = jnp.where(t1 <= hi_g, jnp.int32(1), jnp.int32(0))
        w2 = jnp.where((t2 <= hi_g) & (iota == 0), jnp.int32(1), jnp.int32(0))
        cnt = jnp.sum(w0 + w1 + w2)
        trips = cnt - si + jnp.where(hi_g < end, jnp.int32(1), jnp.int32(0))

        pltpu.make_async_copy(
            atom_hbm.at[pl.ds(off, R), :], fbuf.at[p], sem_f.at[p]).wait()

        @pl.when(g + 2 < nb)
        def _():
            p2 = jnp.where(p == 0, jnp.int32(2), p - 1)
            issue(g + 2, p2)

        def run(_, rc):
            si_r, pos_l, *vr = rc
            sv = vr[:NQ]
            mv_ = vr[NQ:]
            nxt = plsc.load_gather(
                bnd, [jnp.full((L,), si_r + 1, jnp.int32)])[0]
            re_l = jnp.minimum(nxt, hi_g) - off

            def row(r, rcv):
                svi = rcv[:NQ]
                mvi = rcv[NQ:]
                out = []
                for q in range(NQ):
                    f = fbuf[p, r, L * q:L * (q + 1)]
                    out.append(svi[q] + f)
                for q in range(NQ):
                    f = fbuf[p, r, L * q:L * (q + 1)]
                    out.append(jnp.maximum(mvi[q], f))
                return out

            vr2 = lax.fori_loop(pos_l, re_l, row, list(sv) + list(mv_))
            sv2 = vr2[:NQ]
            mv2 = vr2[NQ:]
            ended = nxt <= hi_g

            @pl.when(ended)
            def _():
                for q in range(NQ):
                    acc[si_r, L * q:L * (q + 1)] = sv2[q]
                    acc[si_r, D + L * q:D + L * (q + 1)] = mv2[q]

            si2 = jnp.where(ended, si_r + 1, si_r)
            sv3 = [jnp.where(ended, zero, x) for x in sv2]
            mv3 = [jnp.where(ended, ninf, x) for x in mv2]
            return [si2, re_l] + sv3 + mv3

        rc = lax.fori_loop(0, trips, run,
                           [si, a2] + list(vregs))
        p_next = jnp.where(p == 2, jnp.int32(0), p + 1)
        return [rc[0], p_next] + rc[2:]

    init_carry = [jnp.int32(0), jnp.int32(0)] + [zero] * NQ + [ninf] * NQ
    lax.fori_loop(0, nb, block, init_carry)

    pltpu.sync_copy(acc, out_hbm.at[pl.ds(seg0, SPW), :])


@jax.jit
def _graph_gather(atom_features, mem_padded):
    mesh = plsc.VectorSubcoreMesh(core_axis_name="c", subcore_axis_name="s")
    return pl.kernel(
        _sc_body,
        out_type=jax.ShapeDtypeStruct((B, 2 * D), jnp.float32),
        mesh=mesh,
        compiler_params=pltpu.CompilerParams(
            use_tc_tiling_on_sc=False, needs_layout_passes=False),
        scratch_types=[
            pltpu.VMEM((3, R, D), jnp.float32),
            pltpu.VMEM((3, R + 2 * PAD), jnp.int32),
            pltpu.VMEM((PAD,), jnp.int32),
            pltpu.VMEM((PAD,), jnp.int32),
            pltpu.VMEM((3 * L,), jnp.int32),
            pltpu.VMEM((SPW, 2 * D), jnp.float32),
            pltpu.SemaphoreType.DMA((3,)),
            pltpu.SemaphoreType.DMA((3,)),
            pltpu.SemaphoreType.DMA,
            pltpu.SemaphoreType.DMA,
        ],
    )(atom_features, mem_padded)


def kernel(atom_features, pair_features, membership):
    del pair_features
    return _graph_gather(atom_features, membership.astype(jnp.int32))


# speedup vs baseline: 20.4015x; 1.0007x over previous
import functools

import jax
import jax.numpy as jnp
from jax import lax
from jax.experimental import pallas as pl
from jax.experimental.pallas import tpu as pltpu
from jax.experimental.pallas import tpu_sc as plsc

N = 320000
D = 128
B = 1024
NC, NS, L = 2, 16, 16
NW = NC * NS
SPW = B // NW
R = 256
PAD = 16
NQ = D // L
_NBLK = N // PAD
_BITS = 15
BIG = N + 1


def _cummin(v):
    return -plsc.cummax(-v)


def _sufmin(v):
    return lax.rev(_cummin(lax.rev(v, (0,))), (0,))


_ROUNDS = 5


def _lower_bound2(mem_hbm, sb1, sb2, sem1, sem2, t1, t2):
    iota = lax.iota(jnp.int32, L)

    def probe(c, sbuf, sem):
        lo, sz = c
        pos = lo + lax.div(sz * iota, L)
        rows = jnp.minimum(pos, _NBLK - 1) * PAD
        return pltpu.async_copy(mem_hbm.at[rows], sbuf, sem), pos

    def upd(c, target, sbuf, pos):
        lo, sz = c
        heads = jnp.where(pos >= _NBLK, jnp.int32(B), sbuf[:])
        cnt = jnp.sum(jnp.where(heads < target, jnp.int32(1), jnp.int32(0)))
        lo2 = jnp.where(cnt == 0, lo, lo + lax.div(sz * (cnt - 1), L) + 1)
        hi2 = jnp.where(cnt >= L, lo + sz,
                        jnp.where(cnt == 0, lo, lo + lax.div(sz * cnt, L)))
        return (lo2, hi2 - lo2)

    def body(_, c):
        c1, c2 = c
        d1, pos1 = probe(c1, sb1, sem1)
        d2, pos2 = probe(c2, sb2, sem2)
        d1.wait()
        d2.wait()
        return (upd(c1, t1, sb1, pos1), upd(c2, t2, sb2, pos2))

    init = (jnp.int32(0), jnp.int32(_NBLK))
    (kb1, _), (kb2, _) = lax.fori_loop(0, _ROUNDS, body, (init, init))
    base1 = jnp.maximum(kb1 - 1, 0) * PAD
    base2 = jnp.maximum(kb2 - 1, 0) * PAD
    d1 = pltpu.async_copy(mem_hbm.at[pl.ds(base1, PAD)], sb1, sem1)
    d2 = pltpu.async_copy(mem_hbm.at[pl.ds(base2, PAD)], sb2, sem2)
    d1.wait()
    d2.wait()
    cnt1 = jnp.sum(jnp.where(sb1[:] < t1, jnp.int32(1), jnp.int32(0)))
    cnt2 = jnp.sum(jnp.where(sb2[:] < t2, jnp.int32(1), jnp.int32(0)))
    return base1 + cnt1, base2 + cnt2


def _sc_body(atom_hbm, mem_hbm, out_hbm, fbuf, mbuf, sbuf, sbuf2, bnd, acc,
             sem_f, sem_m, sem_s, sem_s2):
    c = lax.axis_index("c")
    s = lax.axis_index("s")
    wid = s * NC + c
    seg0 = (wid * SPW).astype(jnp.int32)

    start, end = _lower_bound2(mem_hbm, sbuf, sbuf2, sem_s, sem_s2,
                               seg0, seg0 + SPW)

    zero = jnp.zeros((L,), jnp.float32)
    ninf = jnp.full((L,), -jnp.inf, jnp.float32)
    iota = lax.iota(jnp.int32, L)

    def init_seg(i, _):
        for j in range(NQ):
            acc[i, L * j:L * (j + 1)] = zero
            acc[i, D + L * j:D + L * (j + 1)] = ninf
        return 0

    lax.fori_loop(0, SPW, init_seg, 0)

    bnd[0:L] = jnp.full((L,), BIG, jnp.int32)
    bnd[L:2 * L] = jnp.full((L,), BIG, jnp.int32)
    bnd[2 * L:3 * L] = jnp.where(iota == 0, end, BIG)

    n = end - start
    nb = lax.div(n + (R - 1), R)

    def offsets(g):
        lo_g = start + g * R
        off = jnp.minimum(lo_g, N - R)
        moff = lax.div(off, PAD) * PAD - PAD
        moff = jnp.minimum(moff, N - (R + 2 * PAD))
        moff = pl.multiple_of(jnp.maximum(moff, 0), PAD)
        return lo_g, off, moff

    def issue(g, p):
        _, off, moff = offsets(g)
        pltpu.async_copy(atom_hbm.at[pl.ds(off, R), :], fbuf.at[p], sem_f.at[p])
        pltpu.async_copy(mem_hbm.at[pl.ds(moff, R + 2 * PAD)], mbuf.at[p],
                         sem_m.at[p])

    @pl.when(nb > 0)
    def _():
        issue(jnp.int32(0), jnp.int32(0))

    @pl.when(nb > 1)
    def _():
        issue(jnp.int32(1), jnp.int32(1))

    def block(g, carry):
        si, p, *vregs = carry
        lo_g, off, moff = offsets(g)
        hi_g = jnp.minimum(lo_g + R, end)
        pltpu.make_async_copy(
            mem_hbm.at[pl.ds(moff, R + 2 * PAD)], mbuf.at[p], sem_m.at[p]).wait()

        sh = off - moff
        a2 = lo_g - off
        b2 = hi_g - off

        def scan_tile(t, _):
            r0 = t * L
            mv = mbuf[p, pl.ds(sh + r0, L)]
            prev = mbuf[p, pl.ds(jnp.maximum(sh + r0 - 1, 0), L)]
            r_loc = r0 + iota
            mask = ((mv != prev) | (off + r_loc == start)) \
                & (r_loc >= a2) & (r_loc < b2)
            plsc.store_scatter(bnd, [mv - seg0], off + r_loc, mask=mask)
            return 0

        lax.fori_loop(0, R // L, scan_tile, 0)

        t2 = _sufmin(bnd[2 * L:3 * L])
        t1 = jnp.minimum(_sufmin(bnd[L:2 * L]), jnp.broadcast_to(t2[0], (L,)))
        t0 = jnp.minimum(_sufmin(bnd[0:L]), jnp.broadcast_to(t1[0], (L,)))
        bnd[0:L] = t0
        bnd[L:2 * L] = t1
        bnd[2 * L:3 * L] = t2

        w0 = jnp.where((t0 <= hi_g) & (iota >= 1), jnp.int32(1), jnp.int32(0))
        w1 = jnp.where(t1 <= hi_g, jnp.int32(1), jnp.int32(0))
        w2 = jnp.where((t2 <= hi_g) & (iota == 0), jnp.int32(1), jnp.int32(0))
        cnt = jnp.sum(w0 + w1 + w2)
        trips = cnt - si + jnp.where(hi_g < end, jnp.int32(1), jnp.int32(0))

        pltpu.make_async_copy(
            atom_hbm.at[pl.ds(off, R), :], fbuf.at[p], sem_f.at[p]).wait()

        @pl.when(g + 2 < nb)
        def _():
            p2 = jnp.where(p == 0, jnp.int32(2), p - 1)
            issue(g + 2, p2)

        def run(_, rc):
            si_r, pos_l, *vr = rc
            sv = vr[:NQ]
            mv_ = vr[NQ:]
            nxt = plsc.load_gather(
                bnd, [jnp.full((L,), si_r + 1, jnp.int32)])[0]
            re_l = jnp.minimum(nxt, hi_g) - off

            def row(r, rcv):
                svi = rcv[:NQ]
                mvi = rcv[NQ:]
                out = []
                for q in range(NQ):
                    f = fbuf[p, r, L * q:L * (q + 1)]
                    out.append(svi[q] + f)
                for q in range(NQ):
                    f = fbuf[p, r, L * q:L * (q + 1)]
                    out.append(jnp.maximum(mvi[q], f))
                return out

            vr2 = lax.fori_loop(pos_l, re_l, row, list(sv) + list(mv_))
            sv2 = vr2[:NQ]
            mv2 = vr2[NQ:]
            ended = nxt <= hi_g

            @pl.when(ended)
            def _():
                for q in range(NQ):
                    acc[si_r, L * q:L * (q + 1)] = sv2[q]
                    acc[si_r, D + L * q:D + L * (q + 1)] = mv2[q]

            si2 = jnp.where(ended, si_r + 1, si_r)
            sv3 = [jnp.where(ended, zero, x) for x in sv2]
            mv3 = [jnp.where(ended, ninf, x) for x in mv2]
            return [si2, re_l] + sv3 + mv3

        rc = lax.fori_loop(0, trips, run,
                           [si, a2] + list(vregs))
        p_next = jnp.where(p == 2, jnp.int32(0), p + 1)
        return [rc[0], p_next] + rc[2:]

    init_carry = [jnp.int32(0), jnp.int32(0)] + [zero] * NQ + [ninf] * NQ
    lax.fori_loop(0, nb, block, init_carry)

    pltpu.sync_copy(acc, out_hbm.at[pl.ds(seg0, SPW), :])


@jax.jit
def _graph_gather(atom_features, mem_padded):
    mesh = plsc.VectorSubcoreMesh(core_axis_name="c", subcore_axis_name="s")
    return pl.kernel(
        _sc_body,
        out_type=jax.ShapeDtypeStruct((B, 2 * D), jnp.float32),
        mesh=mesh,
        compiler_params=pltpu.CompilerParams(
            use_tc_tiling_on_sc=False, needs_layout_passes=False,
            skip_device_barrier=True),
        scratch_types=[
            pltpu.VMEM((3, R, D), jnp.float32),
            pltpu.VMEM((3, R + 2 * PAD), jnp.int32),
            pltpu.VMEM((PAD,), jnp.int32),
            pltpu.VMEM((PAD,), jnp.int32),
            pltpu.VMEM((3 * L,), jnp.int32),
            pltpu.VMEM((SPW, 2 * D), jnp.float32),
            pltpu.SemaphoreType.DMA((3,)),
            pltpu.SemaphoreType.DMA((3,)),
            pltpu.SemaphoreType.DMA,
            pltpu.SemaphoreType.DMA,
        ],
    )(atom_features, mem_padded)


def kernel(atom_features, pair_features, membership):
    del pair_features
    return _graph_gather(atom_features, membership.astype(jnp.int32))
